# baseline (speedup 1.0000x reference)
import functools

import jax
import jax.numpy as jnp
from jax.experimental import pallas as pl
from jax.experimental.pallas import tpu as pltpu


def _fused_kernel(scale_ref, x_ref, h_ref, w0_ref, b0_ref, w1c_ref, b1c_ref,
                  gv_ref, beta_ref, o_ref, *, pin):
  s = scale_ref[0, 0]
  rows = x_ref.shape[0] // pin
  x = x_ref[...].reshape(rows, pin)
  hh = h_ref[...].reshape(rows, pin)
  pre = x * s + hh
  z = jnp.dot(pre, w0_ref[...], preferred_element_type=jnp.float32)
  y = jnp.maximum(z + b0_ref[...], 0.0)
  cg = jnp.dot(y, w1c_ref[...], preferred_element_type=jnp.float32) + b1c_ref[...]
  var = jnp.dot(cg * cg, gv_ref[...], preferred_element_type=jnp.float32)
  out = cg * jax.lax.rsqrt(var + 1e-5) + beta_ref[...]
  o_ref[...] = out.reshape(rows * pin).astype(o_ref.dtype)


@jax.jit
def _gin_update(literal_embs, h, epsilon, w0, b0, w1, b1, ln_g, ln_b):
  n2, d = literal_embs.shape
  n = n2 // 2
  f32 = jnp.float32
  dh = w0.shape[1]

  p = 1
  if 2 * d < 128 and 128 % (2 * d) == 0 and n % (128 // (2 * d)) == 0:
    p = 128 // (2 * d)
  rows = n // p
  pin, pmid = p * 2 * d, p * 2 * dh

  x1 = literal_embs.reshape(n2 * d)
  h1 = h.reshape(n2 * d)

  w0t, w0b = w0[:d].astype(f32), w0[d:].astype(f32)
  w_pair = jnp.concatenate(
      [jnp.concatenate([w0t, w0b], axis=1),
       jnp.concatenate([w0b, w0t], axis=1)], axis=0)
  eye_p = jnp.eye(p, dtype=f32)
  eye_2p = jnp.eye(2 * p, dtype=f32)
  w0_full = jnp.kron(eye_p, w_pair)

  gamma = ln_g.astype(f32)
  w1f = w1.astype(f32)
  w1c = (w1f - jnp.mean(w1f, axis=1, keepdims=True)) * gamma[None, :]
  b1f = b1.astype(f32)
  b1c = (b1f - jnp.mean(b1f)) * gamma
  w1c_full = jnp.kron(eye_2p, w1c)
  b1c_full = jnp.tile(b1c, 2 * p).reshape(1, pin)

  gv = jnp.full((d, d), 1.0 / d, f32) / (gamma * gamma)[:, None]
  gv_full = jnp.kron(eye_2p, gv)

  b0_full = jnp.tile(b0.astype(f32), 2 * p).reshape(1, pmid)
  beta_full = jnp.tile(ln_b.astype(f32), 2 * p).reshape(1, pin)
  scale = jnp.reshape(jnp.asarray(epsilon, f32) + 1.0, (1, 1))

  tile = 2048 if rows % 2048 == 0 else max(8, (rows // 8) * 8 // 8)
  grid = pl.cdiv(rows, tile)
  blk = tile * pin

  out = pl.pallas_call(
      functools.partial(_fused_kernel, pin=pin),
      out_shape=jax.ShapeDtypeStruct((n2 * d,), literal_embs.dtype),
      grid=(grid,),
      in_specs=[
          pl.BlockSpec(memory_space=pltpu.MemorySpace.SMEM),
          pl.BlockSpec((blk,), lambda i: (i,)),
          pl.BlockSpec((blk,), lambda i: (i,)),
          pl.BlockSpec((pin, pmid), lambda i: (0, 0)),
          pl.BlockSpec((1, pmid), lambda i: (0, 0)),
          pl.BlockSpec((pmid, pin), lambda i: (0, 0)),
          pl.BlockSpec((1, pin), lambda i: (0, 0)),
          pl.BlockSpec((pin, pin), lambda i: (0, 0)),
          pl.BlockSpec((1, pin), lambda i: (0, 0)),
      ],
      out_specs=pl.BlockSpec((blk,), lambda i: (i,)),
      compiler_params=pltpu.CompilerParams(
          dimension_semantics=("parallel",),
          vmem_limit_bytes=64 << 20),
  )(scale, x1, h1, w0_full, b0_full, w1c_full, b1c_full, gv_full, beta_full)
  return out.reshape(n2, d)


def kernel(literal_embs, h, epsilon, w0, b0, w1, b1, ln_g, ln_b):
  return _gin_update(literal_embs, h, epsilon, w0, b0, w1, b1, ln_g, ln_b)
